# baseline (speedup 1.0000x reference)
import functools

import jax
import jax.numpy as jnp
from jax import lax
from jax.experimental import pallas as pl
from jax.experimental.pallas import tpu as pltpu
from jax.experimental.pallas import tpu_sc as plsc

N = 10000
NP = 10240
E = 320000
F = 128
NC, NS = 2, 16
NW = NC * NS
EPW = E // NW
TPN = NP // NS

_mesh = plsc.VectorSubcoreMesh(core_axis_name="c", subcore_axis_name="s")


def _sc_agg_body(B, NB, NBUF, NCH, *refs):
    (h_hbm, src_hbm, dst_hbm, zeros_hbm, acc_hbm, src_idx, dst_idx, rows, acc,
     *sems) = refs
    gsems = sems[:NBUF]
    ssems = sems[NBUF:]
    CB = NB // NCH
    c = lax.axis_index("c")
    s = lax.axis_index("s")
    w = c * NS + s
    base = s * TPN

    pltpu.sync_copy(zeros_hbm.at[pl.ds(base, TPN)], acc.at[pl.ds(base, TPN)])
    plsc.subcore_barrier()

    def fire_g(b, k):
        pltpu.async_copy(h_hbm.at[src_idx.at[b]], rows.at[k], gsems[k])

    def drain_g(b, k):
        pltpu.make_async_copy(h_hbm.at[src_idx.at[b]], rows.at[k],
                              gsems[k]).wait()

    def fire_s(b, k):
        pltpu.async_copy(rows.at[k], acc.at[dst_idx.at[b]], ssems[k], add=True)

    def drain_s(b, k):
        pltpu.make_async_copy(rows.at[k], acc.at[dst_idx.at[b]],
                              ssems[k]).wait()

    for ci in range(NCH):
        if NCH == 1:
            pltpu.sync_copy(src_hbm.at[w], src_idx)
            pltpu.sync_copy(dst_hbm.at[w], dst_idx)
        else:
            pltpu.sync_copy(src_hbm.at[w].at[pl.ds(ci * CB, CB)], src_idx)
            pltpu.sync_copy(dst_hbm.at[w].at[pl.ds(ci * CB, CB)], dst_idx)
        for k in range(2):
            fire_g(k, k)

        def group(i, carry):
            t = i * NBUF
            for k in range(NBUF):
                b = t + k
                kn = (k + 2) % NBUF
                drain_g(b, k)
                fire_s(b, k)

                @pl.when(b >= 3)
                def _():
                    drain_s(b - 3, kn)

                @pl.when(b + 2 <= CB - 1)
                def _():
                    fire_g(b + 2, kn)
            return carry

        lax.fori_loop(0, CB // NBUF, group, 0)
        drain_s(CB - 3, (CB - 3) % NBUF)
        drain_s(CB - 2, (CB - 2) % NBUF)
        drain_s(CB - 1, (CB - 1) % NBUF)

    plsc.subcore_barrier()
    pltpu.sync_copy(acc.at[pl.ds(base, TPN)],
                    acc_hbm.at[c].at[pl.ds(base, TPN)])


def _make_sc_agg(B, NBUF, NCH):
    NB = EPW // B
    CB = NB // NCH
    return pl.kernel(
        functools.partial(_sc_agg_body, B, NB, NBUF, NCH),
        out_type=[jax.ShapeDtypeStruct((NC, NP, F), jnp.float32)],
        mesh=_mesh,
        scratch_types=[
            pltpu.VMEM((CB, B), jnp.int32),
            pltpu.VMEM((CB, B), jnp.int32),
            pltpu.VMEM((NBUF, B, F), jnp.float32),
            pltpu.VMEM_SHARED((NP, F), jnp.float32),
        ] + [pltpu.SemaphoreType.DMA] * (2 * NBUF),
        name="sc_segment_sum",
    )


_sc_agg = _make_sc_agg(50, 5, 5)
_CB = 80


def _sc_cnt_body(dst_hbm, ones_hbm, zeros_hbm, cnt_hbm, dst_idx, ones, cnt, sem):
    NB = EPW // _CB
    c = lax.axis_index("c")
    s = lax.axis_index("s")
    w = c * NS + s
    base = s * TPN

    pltpu.sync_copy(zeros_hbm.at[pl.ds(base, TPN)], cnt.at[pl.ds(base, TPN)])
    pltpu.sync_copy(ones_hbm, ones)
    pltpu.sync_copy(dst_hbm.at[w], dst_idx)
    plsc.subcore_barrier()

    def group(i, carry):
        t = i * 5
        for k in range(5):
            pltpu.async_copy(ones, cnt.at[dst_idx.at[t + k]], sem, add=True)

        @pl.when(i >= 1)
        def _():
            for k in range(5):
                pltpu.make_async_copy(ones, cnt.at[dst_idx.at[t - 5 + k]],
                                      sem).wait()
        return carry

    lax.fori_loop(0, NB // 5, group, 0)
    for k in range(5):
        pltpu.make_async_copy(ones, cnt.at[dst_idx.at[NB - 5 + k]], sem).wait()
    plsc.subcore_barrier()

    pltpu.sync_copy(cnt.at[pl.ds(base, TPN)],
                    cnt_hbm.at[c].at[pl.ds(base, TPN)])


_sc_cnt = pl.kernel(
    _sc_cnt_body,
    out_type=[jax.ShapeDtypeStruct((NC, NP, F), jnp.float32)],
    mesh=_mesh,
    scratch_types=[
        pltpu.VMEM((EPW // _CB, _CB), jnp.int32),
        pltpu.VMEM((_CB, F), jnp.float32),
        pltpu.VMEM_SHARED((NP, F), jnp.float32),
        pltpu.SemaphoreType.DMA,
    ],
    name="sc_segment_cnt",
)


def _tc_layer_body(final, parts_ref, cntp_ref, h_ref, wl_ref, wr_ref, b_ref,
                   o_ref):
    p = parts_ref[0] + parts_ref[1]
    cnt_lanes = jnp.sum(cntp_ref[0] + cntp_ref[1], axis=-1)
    inv = 128.0 / jnp.clip(cnt_lanes, 128.0, None)
    agg = p * inv[:, None]
    y = (jnp.dot(agg, wl_ref[...], preferred_element_type=jnp.float32)
         + jnp.dot(h_ref[...], wr_ref[...], preferred_element_type=jnp.float32)
         + b_ref[...][None, :])
    if not final:
        o_ref[...] = jnp.maximum(y, 0.0)
    else:
        ncls = o_ref.shape[1]
        col = lax.broadcasted_iota(jnp.int32, y.shape, 1)
        valid = col < ncls
        masked = jnp.where(valid, y, -jnp.inf)
        m = jnp.max(masked, axis=1, keepdims=True)
        lse = m + jnp.log(jnp.sum(jnp.where(valid, jnp.exp(masked - m), 0.0),
                                  axis=1, keepdims=True))
        o_ref[...] = (y - lse)[:, :ncls]


def _tc_layer(parts, cntp, h, wl, wr, b, final, ncols=F):
    bm = 1000
    return pl.pallas_call(
        functools.partial(_tc_layer_body, final),
        grid=(N // bm,),
        in_specs=[
            pl.BlockSpec((NC, bm, F), lambda i: (0, i, 0)),
            pl.BlockSpec((NC, bm, F), lambda i: (0, i, 0)),
            pl.BlockSpec((bm, F), lambda i: (i, 0)),
            pl.BlockSpec((F, F), lambda i: (0, 0)),
            pl.BlockSpec((F, F), lambda i: (0, 0)),
            pl.BlockSpec((F,), lambda i: (0,)),
        ],
        out_specs=pl.BlockSpec((bm, ncols), lambda i: (i, 0)),
        out_shape=jax.ShapeDtypeStruct((N, ncols), jnp.float32),
    )(parts, cntp, h, wl, wr, b)


def kernel(x, edge_index, Wl0, Wr0, b0, Wl1, Wr1, b1, Wl2, Wr2, b2):
    src = edge_index[0].astype(jnp.int32)
    dst = edge_index[1].astype(jnp.int32)
    B0 = 50
    src2 = src.reshape(NW, EPW // B0, B0)
    dst2 = dst.reshape(NW, EPW // B0, B0)
    dst2b = dst.reshape(NW, EPW // _CB, _CB)

    C = Wl2.shape[1]
    Wl2p = jnp.zeros((F, F), jnp.float32).at[:, :C].set(Wl2)
    Wr2p = jnp.zeros((F, F), jnp.float32).at[:, :C].set(Wr2)
    b2p = jnp.zeros((F,), jnp.float32).at[:C].set(b2)

    zeros = jnp.zeros((NP, F), jnp.float32)
    (cntp,) = _sc_cnt(dst2b, jnp.ones((_CB, F), jnp.float32), zeros)
    (parts0,) = _sc_agg(x, src2, dst2, zeros)
    h1 = _tc_layer(parts0, cntp, x, Wl0, Wr0, b0, final=False)
    (parts1,) = _sc_agg(h1, src2, dst2, zeros)
    h2 = _tc_layer(parts1, cntp, h1, Wl1, Wr1, b1, final=False)
    (parts2,) = _sc_agg(h2, src2, dst2, zeros)
    return _tc_layer(parts2, cntp, h2, Wl2p, Wr2p, b2p, final=True, ncols=C)


# speedup vs baseline: 1.1385x; 1.1385x over previous
import functools

import jax
import jax.numpy as jnp
from jax import lax
from jax.experimental import pallas as pl
from jax.experimental.pallas import tpu as pltpu
from jax.experimental.pallas import tpu_sc as plsc

N = 10000
NP = 10240
E = 320000
F = 128
NC, NS = 2, 16
NW = NC * NS
EPW = E // NW
TPN = NP // NS

_mesh = plsc.VectorSubcoreMesh(core_axis_name="c", subcore_axis_name="s")


def _sc_agg_body(B, NB, NBUF, NCH, *refs):
    (h_hbm, src_hbm, dst_hbm, zeros_hbm, acc_hbm, src_idx, dst_idx, rows, acc,
     *sems) = refs
    gsems = sems[:NBUF]
    ssems = sems[NBUF:]
    CB = NB // NCH
    c = lax.axis_index("c")
    s = lax.axis_index("s")
    w = c * NS + s
    base = s * TPN

    pltpu.sync_copy(zeros_hbm.at[pl.ds(base, TPN)], acc.at[pl.ds(base, TPN)])
    plsc.subcore_barrier()

    def fire_g(b, k):
        pltpu.async_copy(h_hbm.at[src_idx.at[b]], rows.at[k], gsems[k])

    def drain_g(b, k):
        pltpu.make_async_copy(h_hbm.at[src_idx.at[b]], rows.at[k],
                              gsems[k]).wait()

    def fire_s(b, k):
        pltpu.async_copy(rows.at[k], acc.at[dst_idx.at[b]], ssems[k], add=True)

    def drain_s(b, k):
        pltpu.make_async_copy(rows.at[k], acc.at[dst_idx.at[b]],
                              ssems[k]).wait()

    for ci in range(NCH):
        if NCH == 1:
            pltpu.sync_copy(src_hbm.at[w], src_idx)
            pltpu.sync_copy(dst_hbm.at[w], dst_idx)
        else:
            pltpu.sync_copy(src_hbm.at[w].at[pl.ds(ci * CB, CB)], src_idx)
            pltpu.sync_copy(dst_hbm.at[w].at[pl.ds(ci * CB, CB)], dst_idx)
        for k in range(3):
            fire_g(k, k)

        def group(i, carry):
            t = i * NBUF
            for k in range(NBUF):
                b = t + k
                kn = (k + 3) % NBUF
                drain_g(b, k)
                fire_s(b, k)

                @pl.when(b >= 2)
                def _():
                    drain_s(b - 2, kn)

                @pl.when(b + 3 <= CB - 1)
                def _():
                    fire_g(b + 3, kn)
            return carry

        lax.fori_loop(0, CB // NBUF, group, 0)
        drain_s(CB - 2, (CB - 2) % NBUF)
        drain_s(CB - 1, (CB - 1) % NBUF)

    plsc.subcore_barrier()
    pltpu.sync_copy(acc.at[pl.ds(base, TPN)],
                    acc_hbm.at[c].at[pl.ds(base, TPN)])


def _make_sc_agg(B, NBUF, NCH):
    NB = EPW // B
    CB = NB // NCH
    return pl.kernel(
        functools.partial(_sc_agg_body, B, NB, NBUF, NCH),
        out_type=[jax.ShapeDtypeStruct((NC, NP, F), jnp.float32)],
        mesh=_mesh,
        scratch_types=[
            pltpu.VMEM((CB, B), jnp.int32),
            pltpu.VMEM((CB, B), jnp.int32),
            pltpu.VMEM((NBUF, B, F), jnp.float32),
            pltpu.VMEM_SHARED((NP, F), jnp.float32),
        ] + [pltpu.SemaphoreType.DMA] * (2 * NBUF),
        name="sc_segment_sum",
    )


_sc_agg = _make_sc_agg(50, 5, 5)
_CB = 80


def _sc_cnt_body(dst_hbm, ones_hbm, zeros_hbm, cnt_hbm, dst_idx, ones, cnt, sem):
    NB = EPW // _CB
    c = lax.axis_index("c")
    s = lax.axis_index("s")
    w = c * NS + s
    base = s * TPN

    pltpu.sync_copy(zeros_hbm.at[pl.ds(base, TPN)], cnt.at[pl.ds(base, TPN)])
    pltpu.sync_copy(ones_hbm, ones)
    pltpu.sync_copy(dst_hbm.at[w], dst_idx)
    plsc.subcore_barrier()

    def group(i, carry):
        t = i * 5
        for k in range(5):
            pltpu.async_copy(ones, cnt.at[dst_idx.at[t + k]], sem, add=True)

        @pl.when(i >= 1)
        def _():
            for k in range(5):
                pltpu.make_async_copy(ones, cnt.at[dst_idx.at[t - 5 + k]],
                                      sem).wait()
        return carry

    lax.fori_loop(0, NB // 5, group, 0)
    for k in range(5):
        pltpu.make_async_copy(ones, cnt.at[dst_idx.at[NB - 5 + k]], sem).wait()
    plsc.subcore_barrier()

    pltpu.sync_copy(cnt.at[pl.ds(base, TPN)],
                    cnt_hbm.at[c].at[pl.ds(base, TPN)])


_sc_cnt = pl.kernel(
    _sc_cnt_body,
    out_type=[jax.ShapeDtypeStruct((NC, NP, F), jnp.float32)],
    mesh=_mesh,
    scratch_types=[
        pltpu.VMEM((EPW // _CB, _CB), jnp.int32),
        pltpu.VMEM((_CB, F), jnp.float32),
        pltpu.VMEM_SHARED((NP, F), jnp.float32),
        pltpu.SemaphoreType.DMA,
    ],
    name="sc_segment_cnt",
)


def _tc_layer_body(final, parts_ref, cntp_ref, h_ref, wl_ref, wr_ref, b_ref,
                   o_ref):
    p = parts_ref[0] + parts_ref[1]
    cnt_lanes = jnp.sum(cntp_ref[0] + cntp_ref[1], axis=-1)
    inv = 128.0 / jnp.clip(cnt_lanes, 128.0, None)
    agg = p * inv[:, None]
    y = (jnp.dot(agg, wl_ref[...], preferred_element_type=jnp.float32)
         + jnp.dot(h_ref[...], wr_ref[...], preferred_element_type=jnp.float32)
         + b_ref[...][None, :])
    if not final:
        o_ref[...] = jnp.maximum(y, 0.0)
    else:
        ncls = o_ref.shape[1]
        col = lax.broadcasted_iota(jnp.int32, y.shape, 1)
        valid = col < ncls
        masked = jnp.where(valid, y, -jnp.inf)
        m = jnp.max(masked, axis=1, keepdims=True)
        lse = m + jnp.log(jnp.sum(jnp.where(valid, jnp.exp(masked - m), 0.0),
                                  axis=1, keepdims=True))
        o_ref[...] = (y - lse)[:, :ncls]


def _tc_layer(parts, cntp, h, wl, wr, b, final, ncols=F):
    bm = 1000
    return pl.pallas_call(
        functools.partial(_tc_layer_body, final),
        grid=(N // bm,),
        in_specs=[
            pl.BlockSpec((NC, bm, F), lambda i: (0, i, 0)),
            pl.BlockSpec((NC, bm, F), lambda i: (0, i, 0)),
            pl.BlockSpec((bm, F), lambda i: (i, 0)),
            pl.BlockSpec((F, F), lambda i: (0, 0)),
            pl.BlockSpec((F, F), lambda i: (0, 0)),
            pl.BlockSpec((F,), lambda i: (0,)),
        ],
        out_specs=pl.BlockSpec((bm, ncols), lambda i: (i, 0)),
        out_shape=jax.ShapeDtypeStruct((N, ncols), jnp.float32),
    )(parts, cntp, h, wl, wr, b)


def kernel(x, edge_index, Wl0, Wr0, b0, Wl1, Wr1, b1, Wl2, Wr2, b2):
    src = edge_index[0].astype(jnp.int32)
    dst = edge_index[1].astype(jnp.int32)
    B0 = 50
    src2 = src.reshape(NW, EPW // B0, B0)
    dst2 = dst.reshape(NW, EPW // B0, B0)
    dst2b = dst.reshape(NW, EPW // _CB, _CB)

    C = Wl2.shape[1]
    Wl2p = jnp.zeros((F, F), jnp.float32).at[:, :C].set(Wl2)
    Wr2p = jnp.zeros((F, F), jnp.float32).at[:, :C].set(Wr2)
    b2p = jnp.zeros((F,), jnp.float32).at[:C].set(b2)

    zeros = jnp.zeros((NP, F), jnp.float32)
    (cntp,) = _sc_cnt(dst2b, jnp.ones((_CB, F), jnp.float32), zeros)
    (parts0,) = _sc_agg(x, src2, dst2, zeros)
    h1 = _tc_layer(parts0, cntp, x, Wl0, Wr0, b0, final=False)
    (parts1,) = _sc_agg(h1, src2, dst2, zeros)
    h2 = _tc_layer(parts1, cntp, h1, Wl1, Wr1, b1, final=False)
    (parts2,) = _sc_agg(h2, src2, dst2, zeros)
    return _tc_layer(parts2, cntp, h2, Wl2p, Wr2p, b2p, final=True, ncols=C)
